# baseline (speedup 1.0000x reference)
import math

import jax
import jax.numpy as jnp
from jax.experimental import pallas as pl

B, N, T = 4, 1024, 96
D, H, NLAYERS = 64, 8, 2
PATCH, STRIDE = 4, 2
L = (T - PATCH) // STRIDE + 1
LP = 48
K, TAU, PRED = 16, 0.5, 24
DH = D // H
RBLK = 128
NB = N // RBLK

_i32 = jnp.int32
_f32 = jnp.float32


def _gather_body(obsr_ref, x_ref, ve_ref, xg_ref, veg_ref, nobs_ref):
    o = obsr_ref[0]
    c = o
    s = 1
    while s < N:
        c = c + jnp.concatenate(
            [jnp.zeros((1, s), _f32), c[:, :N - s]], axis=1)
        s *= 2
    nob = c[:, N - 1:N]
    iota_r = jax.lax.broadcasted_iota(_i32, (1, N), 1).astype(_f32)
    pos = jnp.where(o > 0.0, c - 1.0, nob + iota_r - c)
    pio = jax.lax.broadcasted_iota(_i32, (N, N), 0)
    P = (pio == pos.astype(_i32)).astype(_f32)
    xg_ref[0] = P @ x_ref[0]
    veg_ref[0] = P @ ve_ref[...]
    nobs_ref[0] = nob


def _mixer_body(pt_ref, veg_ref, nobs_ref, Wp_ref, bp_ref, Wt1_ref, ck_ref,
                Wt2_ref, Wh_ref, tok_ref, y_ref):
    nob_i = nobs_ref[0, 0, 0].astype(_i32)

    @pl.when(pl.program_id(1) * RBLK < nob_i)
    def _():
        pt = pt_ref[0].reshape(LP * RBLK, PATCH)
        h = pt @ Wp_ref[...] + bp_ref[...]
        u = (h @ Wt1_ref[...]).reshape(LP, RBLK, 2 * D)
        c0 = ck_ref[0:1, :][None]
        c1 = ck_ref[1:2, :][None]
        c2 = ck_ref[2:3, :][None]
        c3 = ck_ref[3:4, :][None]
        v = (u * c3
             + jnp.concatenate([jnp.zeros((1, RBLK, 2 * D), _f32),
                                u[:LP - 1]], axis=0) * c2
             + jnp.concatenate([jnp.zeros((2, RBLK, 2 * D), _f32),
                                u[:LP - 2]], axis=0) * c1
             + jnp.concatenate([jnp.zeros((3, RBLK, 2 * D), _f32),
                                u[:LP - 3]], axis=0) * c0)
        sil = (v * jax.nn.sigmoid(v)).reshape(LP * RBLK, 2 * D)
        h2 = (h + sil @ Wt2_ref[...]).reshape(LP, RBLK, D)
        tok_ref[0] = (jnp.sum(h2[:L], axis=0) * _f32(1.0 / L) + veg_ref[0])
        acc = h2[0] @ Wh_ref[0:D, :]
        for l in range(1, L):
            acc = acc + h2[l] @ Wh_ref[l * D:(l + 1) * D, :]
        y_ref[0] = acc


_KD = (((1,), (1,)), ((), ()))


KB = 256


def _enc_body(tok_ref, nobs_ref, Wq_ref, Wk_ref, Wv_ref, Wo_ref, out_ref):
    nob = nobs_ref[0, 0, 0]
    nob_i = nob.astype(_i32)
    qb = pl.program_id(1)

    @pl.when(qb * RBLK < nob_i)
    def _():
        tq = tok_ref[0, pl.ds(qb * RBLK, RBLK), :]
        q = tq @ Wq_ref[...]
        s1 = _f32(1.0 / math.sqrt(float(DH)))
        nkb = (nob_i + (KB - 1)) // KB

        def body(kb, carry):
            ms, ss, os_ = carry
            kblk = tok_ref[0, pl.ds(kb * KB, KB), :]
            ciok = jax.lax.broadcasted_iota(_i32, (KB, 1), 0) + kb * KB
            kblk = jnp.where(ciok < nob_i, kblk, 0.0)
            kk = kblk @ Wk_ref[...]
            vv = kblk @ Wv_ref[...]
            rio = jax.lax.broadcasted_iota(_i32, (1, KB), 1) + kb * KB
            msk = rio < nob_i
            nms, nss, nos = [], [], []
            for hh in range(H):
                qh = q[:, hh * DH:(hh + 1) * DH]
                kh = kk[:, hh * DH:(hh + 1) * DH]
                vh = vv[:, hh * DH:(hh + 1) * DH]
                sc = jnp.where(msk, jax.lax.dot_general(qh, kh, _KD) * s1,
                               _f32(-1e9))
                bm = jnp.max(sc, axis=-1, keepdims=True)
                mh = jnp.maximum(ms[hh], bm)
                p = jnp.exp(sc - mh)
                corr = jnp.exp(ms[hh] - mh)
                nms.append(mh)
                nss.append(ss[hh] * corr + jnp.sum(p, axis=-1, keepdims=True))
                nos.append(os_[hh] * corr + p @ vh)
            return tuple(nms), tuple(nss), tuple(nos)

        minit = tuple(jnp.full((RBLK, 1), -1e30, _f32) for _ in range(H))
        sinit = tuple(jnp.zeros((RBLK, 1), _f32) for _ in range(H))
        oinit = tuple(jnp.zeros((RBLK, DH), _f32) for _ in range(H))
        ms, ss, os_ = jax.lax.fori_loop(0, nkb, body, (minit, sinit, oinit))
        outs = [os_[hh] / ss[hh] for hh in range(H)]
        out_ref[0] = tq + jnp.concatenate(outs, axis=1) @ Wo_ref[...]


def _dec_body(tok_ref, veg_ref, nobs_ref, Wq2_ref, Wk2_ref, Wv2_ref, Wo2_ref,
              C_ref, CT_ref, Whs_ref, ydec_ref, qsub_ref, wsub_ref):
    nob = nobs_ref[0, 0, 0]
    nob_i = nob.astype(_i32)
    qb = pl.program_id(1)
    tokf = tok_ref[0]
    cio = jax.lax.broadcasted_iota(_i32, (N, 1), 0)
    tokm = jnp.where(cio < nob_i, tokf, 0.0)
    qsub = jnp.sum(tokm, axis=0, keepdims=True) / nob
    CT = CT_ref[...]
    cn2 = jnp.sum(CT * CT, axis=0, keepdims=True)
    logits = (2.0 * (qsub @ CT) - cn2) * _f32(1.0 / TAU)
    m = jnp.max(logits, axis=-1, keepdims=True)
    e = jnp.exp(logits - m)
    wsub = e / jnp.sum(e, axis=-1, keepdims=True)

    @pl.when(qb == NB - 1)
    def _():
        qsub_ref[0] = qsub
        wsub_ref[0] = wsub

    @pl.when((qb + 1) * RBLK > nob_i)
    def _():
        mt = veg_ref[0] + (wsub @ C_ref[...])
        q2 = mt @ Wq2_ref[...]
        s2c = _f32(1.0 / math.sqrt(float(D)))
        nkb = (nob_i + (KB - 1)) // KB

        def body(kb, carry):
            m2, s2, o2 = carry
            kblk = tok_ref[0, pl.ds(kb * KB, KB), :]
            ciok = jax.lax.broadcasted_iota(_i32, (KB, 1), 0) + kb * KB
            kblk = jnp.where(ciok < nob_i, kblk, 0.0)
            k2 = kblk @ Wk2_ref[...]
            v2 = kblk @ Wv2_ref[...]
            rio = jax.lax.broadcasted_iota(_i32, (1, KB), 1) + kb * KB
            sc2 = jnp.where(rio < nob_i,
                            jax.lax.dot_general(q2, k2, _KD) * s2c,
                            _f32(-1e9))
            bm = jnp.max(sc2, axis=-1, keepdims=True)
            mn = jnp.maximum(m2, bm)
            p = jnp.exp(sc2 - mn)
            corr = jnp.exp(m2 - mn)
            return (mn, s2 * corr + jnp.sum(p, axis=-1, keepdims=True),
                    o2 * corr + p @ v2)

        m2, s2, o2 = jax.lax.fori_loop(
            0, nkb, body,
            (jnp.full((RBLK, 1), -1e30, _f32), jnp.zeros((RBLK, 1), _f32),
             jnp.zeros((RBLK, D), _f32)))
        mo = mt + (o2 / s2) @ Wo2_ref[...]
        ydec_ref[0] = mo @ Whs_ref[...]


def _fin_body(obsc_ref, ymix_ref, ydec_ref, bh_ref, y_ref):
    oc = obsc_ref[0]
    c = oc
    s = 1
    while s < N:
        c = c + jnp.concatenate(
            [jnp.zeros((s, 1), _f32), c[:N - s]], axis=0)
        s *= 2
    nob = c[N - 1:N, :]
    cio = jax.lax.broadcasted_iota(_i32, (N, 1), 0).astype(_f32)
    posT = jnp.where(oc > 0.0, c - 1.0, nob + cio - c)
    rio = jax.lax.broadcasted_iota(_i32, (N, N), 1)
    PT = (rio == posT.astype(_i32)).astype(_f32)
    sel = cio < nob
    yc = jnp.where(sel, ymix_ref[0], ydec_ref[0])
    y_ref[0] = PT @ yc + bh_ref[...]


def kernel(x_full, obs_mask, W_patch, b_patch, Wt1, conv_k, Wt2, var_emb,
           Wq, Wk, Wv, Wo, Wq2, Wk2, Wv2, Wo2, C, W_head, b_head):
    obsf = obs_mask.astype(_f32)
    obs_col = obsf.reshape(B, N, 1)
    obs_row = obsf.reshape(B, 1, N)
    bp2 = b_patch.reshape(1, D)
    ckT = conv_k.T
    bh2 = b_head.reshape(1, PRED)
    CT = C.T
    Whs = W_head.reshape(L, D, PRED).sum(axis=0)

    g2 = lambda b: (0, 0)
    g3 = lambda b: (b, 0, 0)
    xg, veg, nobs = pl.pallas_call(
        _gather_body,
        grid=(B,),
        in_specs=[
            pl.BlockSpec((1, 1, N), g3),
            pl.BlockSpec((1, N, T), g3),
            pl.BlockSpec((N, D), g2),
        ],
        out_specs=[
            pl.BlockSpec((1, N, T), g3),
            pl.BlockSpec((1, N, D), g3),
            pl.BlockSpec((1, 1, 1), g3),
        ],
        out_shape=[
            jax.ShapeDtypeStruct((B, N, T), _f32),
            jax.ShapeDtypeStruct((B, N, D), _f32),
            jax.ShapeDtypeStruct((B, 1, 1), _f32),
        ],
    )(obs_row, x_full, var_emb)

    x_pad = jnp.pad(xg, ((0, 0), (0, 0), (0, 2)))
    widx = jnp.arange(LP)[:, None] * STRIDE + jnp.arange(PATCH)[None, :]
    patches_tm = x_pad[:, :, widx].transpose(0, 2, 1, 3)

    w2 = lambda b, n: (0, 0)
    nb3 = lambda b, n: (b, 0, 0)
    tok, y_mix = pl.pallas_call(
        _mixer_body,
        grid=(B, NB),
        in_specs=[
            pl.BlockSpec((1, LP, RBLK, PATCH), lambda b, n: (b, 0, n, 0)),
            pl.BlockSpec((1, RBLK, D), lambda b, n: (b, n, 0)),
            pl.BlockSpec((1, 1, 1), nb3),
            pl.BlockSpec((PATCH, D), w2),
            pl.BlockSpec((1, D), w2),
            pl.BlockSpec((D, 2 * D), w2),
            pl.BlockSpec((PATCH, 2 * D), w2),
            pl.BlockSpec((2 * D, D), w2),
            pl.BlockSpec((L * D, PRED), w2),
        ],
        out_specs=[
            pl.BlockSpec((1, RBLK, D), lambda b, n: (b, n, 0)),
            pl.BlockSpec((1, RBLK, PRED), lambda b, n: (b, n, 0)),
        ],
        out_shape=[
            jax.ShapeDtypeStruct((B, N, D), _f32),
            jax.ShapeDtypeStruct((B, N, PRED), _f32),
        ],
    )(patches_tm, veg, nobs, W_patch, bp2, Wt1, ckT, Wt2, W_head)

    for lyr in range(NLAYERS):
        tok = pl.pallas_call(
            _enc_body,
            grid=(B, NB),
            in_specs=[
                pl.BlockSpec((1, N, D), nb3),
                pl.BlockSpec((1, 1, 1), nb3),
                pl.BlockSpec((D, D), w2),
                pl.BlockSpec((D, D), w2),
                pl.BlockSpec((D, D), w2),
                pl.BlockSpec((D, D), w2),
            ],
            out_specs=pl.BlockSpec((1, RBLK, D), lambda b, n: (b, n, 0)),
            out_shape=jax.ShapeDtypeStruct((B, N, D), _f32),
        )(tok, nobs, Wq[lyr], Wk[lyr], Wv[lyr], Wo[lyr])

    y_dec, q_sub, w_sub = pl.pallas_call(
        _dec_body,
        grid=(B, NB),
        in_specs=[
            pl.BlockSpec((1, N, D), nb3),
            pl.BlockSpec((1, RBLK, D), lambda b, n: (b, n, 0)),
            pl.BlockSpec((1, 1, 1), nb3),
            pl.BlockSpec((D, D), w2),
            pl.BlockSpec((D, D), w2),
            pl.BlockSpec((D, D), w2),
            pl.BlockSpec((D, D), w2),
            pl.BlockSpec((K, D), w2),
            pl.BlockSpec((D, K), w2),
            pl.BlockSpec((D, PRED), w2),
        ],
        out_specs=[
            pl.BlockSpec((1, RBLK, PRED), lambda b, n: (b, n, 0)),
            pl.BlockSpec((1, 1, D), nb3),
            pl.BlockSpec((1, 1, K), nb3),
        ],
        out_shape=[
            jax.ShapeDtypeStruct((B, N, PRED), _f32),
            jax.ShapeDtypeStruct((B, 1, D), _f32),
            jax.ShapeDtypeStruct((B, 1, K), _f32),
        ],
    )(tok, veg, nobs, Wq2, Wk2, Wv2, Wo2, C, CT, Whs)

    y_hat = pl.pallas_call(
        _fin_body,
        grid=(B,),
        in_specs=[
            pl.BlockSpec((1, N, 1), g3),
            pl.BlockSpec((1, N, PRED), g3),
            pl.BlockSpec((1, N, PRED), g3),
            pl.BlockSpec((1, PRED), g2),
        ],
        out_specs=pl.BlockSpec((1, N, PRED), g3),
        out_shape=jax.ShapeDtypeStruct((B, N, PRED), _f32),
    )(obs_col, y_mix, y_dec, bh2)

    return (y_hat, q_sub.reshape(B, D), w_sub.reshape(B, K))


# speedup vs baseline: 1.1537x; 1.1537x over previous
import functools
import math

import jax
import jax.numpy as jnp
from jax import lax
from jax.experimental import pallas as pl
from jax.experimental.pallas import tpu as pltpu
from jax.experimental.pallas import tpu_sc as plsc

B, N, T = 4, 1024, 96
D, H, NLAYERS = 64, 8, 2
PATCH, STRIDE = 4, 2
L = (T - PATCH) // STRIDE + 1
LP = 48
K, TAU, PRED = 16, 0.5, 24
DH = D // H
RBLK = 128
NB = N // RBLK

_i32 = jnp.int32
_f32 = jnp.float32


def _route_body(obsr_ref, posg_ref, nobs_ref):
    o = obsr_ref[0]
    c = o
    s = 1
    while s < N:
        c = c + jnp.concatenate(
            [jnp.zeros((1, s), _f32), c[:, :N - s]], axis=1)
        s *= 2
    nob = c[:, N - 1:N]
    iota_r = jax.lax.broadcasted_iota(_i32, (1, N), 1).astype(_f32)
    pos = jnp.where(o > 0.0, c - 1.0, nob + iota_r - c)
    posg_ref[0] = pos.astype(_i32) + pl.program_id(0) * N
    nobs_ref[0] = nob


_SC_NC, _SC_NW = 2, 32
_ROWS = B * N
_CW = 256
_WPER = _ROWS // _SC_NW


def _sc_scatter_body(xve_hbm, posg_hbm, out_hbm, idx_v, rows_v, sem):
    wid = lax.axis_index("s") * _SC_NC + lax.axis_index("c")
    base = wid * _WPER
    pltpu.sync_copy(posg_hbm.at[pl.ds(base, _WPER)], idx_v)
    pltpu.sync_copy(xve_hbm.at[pl.ds(base, _WPER)], rows_v)
    pltpu.async_copy(rows_v, out_hbm.at[idx_v], sem).wait()


def _mixer_body(pt_ref, veg_ref, nobs_ref, Wp_ref, bp_ref, Wt1_ref, ck_ref,
                Wt2_ref, Wh_ref, tok_ref, y_ref):
    nob_i = nobs_ref[0, 0, 0].astype(_i32)

    @pl.when(pl.program_id(1) * RBLK < nob_i)
    def _():
        pt = pt_ref[0].reshape(LP * RBLK, PATCH)
        h = pt @ Wp_ref[...] + bp_ref[...]
        u = (h @ Wt1_ref[...]).reshape(LP, RBLK, 2 * D)
        c0 = ck_ref[0:1, :][None]
        c1 = ck_ref[1:2, :][None]
        c2 = ck_ref[2:3, :][None]
        c3 = ck_ref[3:4, :][None]
        v = (u * c3
             + jnp.concatenate([jnp.zeros((1, RBLK, 2 * D), _f32),
                                u[:LP - 1]], axis=0) * c2
             + jnp.concatenate([jnp.zeros((2, RBLK, 2 * D), _f32),
                                u[:LP - 2]], axis=0) * c1
             + jnp.concatenate([jnp.zeros((3, RBLK, 2 * D), _f32),
                                u[:LP - 3]], axis=0) * c0)
        sil = (v * jax.nn.sigmoid(v)).reshape(LP * RBLK, 2 * D)
        h2 = (h + sil @ Wt2_ref[...]).reshape(LP, RBLK, D)
        tok_ref[0] = (jnp.sum(h2[:L], axis=0) * _f32(1.0 / L) + veg_ref[0])
        acc = h2[0] @ Wh_ref[0:D, :]
        for l in range(1, L):
            acc = acc + h2[l] @ Wh_ref[l * D:(l + 1) * D, :]
        y_ref[0] = acc


_KD = (((1,), (1,)), ((), ()))


def _enc_body(tok_ref, nobs_ref, Wq_ref, Wk_ref, Wv_ref, Wo_ref, out_ref):
    nob = nobs_ref[0, 0, 0]
    nob_i = nob.astype(_i32)
    qb = pl.program_id(1)

    @pl.when(qb * RBLK < nob_i)
    def _():
        tokf = tok_ref[0]
        cio = jax.lax.broadcasted_iota(_i32, (N, 1), 0)
        tokm = jnp.where(cio < nob_i, tokf, 0.0)
        tq = tok_ref[0, pl.ds(qb * RBLK, RBLK), :]
        q = tq @ Wq_ref[...]
        kk = tokm @ Wk_ref[...]
        vv = tokm @ Wv_ref[...]
        rio = jax.lax.broadcasted_iota(_i32, (1, N), 1)
        bias = jnp.where(rio < nob_i, 0.0, -1e9).astype(_f32)
        s1 = _f32(1.0 / math.sqrt(float(DH)))
        outs = []
        for hh in range(H):
            qh = q[:, hh * DH:(hh + 1) * DH]
            kh = kk[:, hh * DH:(hh + 1) * DH]
            vh = vv[:, hh * DH:(hh + 1) * DH]
            sc = jax.lax.dot_general(qh, kh, _KD) * s1 + bias
            m = jnp.max(sc, axis=-1, keepdims=True)
            e = jnp.exp(sc - m)
            ssum = jnp.sum(e, axis=-1, keepdims=True)
            outs.append((e @ vh) / ssum)
        out_ref[0] = tq + jnp.concatenate(outs, axis=1) @ Wo_ref[...]


def _dec_body(tok_ref, veg_ref, nobs_ref, Wq2_ref, Wk2_ref, Wv2_ref, Wo2_ref,
              C_ref, CT_ref, Whs_ref, ydec_ref, qsub_ref, wsub_ref):
    nob = nobs_ref[0, 0, 0]
    nob_i = nob.astype(_i32)
    qb = pl.program_id(1)
    tokf = tok_ref[0]
    cio = jax.lax.broadcasted_iota(_i32, (N, 1), 0)
    tokm = jnp.where(cio < nob_i, tokf, 0.0)
    qsub = jnp.sum(tokm, axis=0, keepdims=True) / nob
    CT = CT_ref[...]
    cn2 = jnp.sum(CT * CT, axis=0, keepdims=True)
    logits = (2.0 * (qsub @ CT) - cn2) * _f32(1.0 / TAU)
    m = jnp.max(logits, axis=-1, keepdims=True)
    e = jnp.exp(logits - m)
    wsub = e / jnp.sum(e, axis=-1, keepdims=True)

    @pl.when(qb == NB - 1)
    def _():
        qsub_ref[0] = qsub
        wsub_ref[0] = wsub

    @pl.when((qb + 1) * RBLK > nob_i)
    def _():
        mt = veg_ref[0] + (wsub @ C_ref[...])
        q2 = mt @ Wq2_ref[...]
        k2 = tokm @ Wk2_ref[...]
        v2 = tokm @ Wv2_ref[...]
        rio = jax.lax.broadcasted_iota(_i32, (1, N), 1)
        bias = jnp.where(rio < nob_i, 0.0, -1e9).astype(_f32)
        sc2 = (jax.lax.dot_general(q2, k2, _KD) * _f32(1.0 / math.sqrt(float(D)))
               + bias)
        m2 = jnp.max(sc2, axis=-1, keepdims=True)
        e2 = jnp.exp(sc2 - m2)
        s2 = jnp.sum(e2, axis=-1, keepdims=True)
        mo = mt + ((e2 @ v2) / s2) @ Wo2_ref[...]
        ydec_ref[0] = mo @ Whs_ref[...]


def _fin_body(obsc_ref, ymix_ref, ydec_ref, bh_ref, y_ref):
    oc = obsc_ref[0]
    c = oc
    s = 1
    while s < N:
        c = c + jnp.concatenate(
            [jnp.zeros((s, 1), _f32), c[:N - s]], axis=0)
        s *= 2
    nob = c[N - 1:N, :]
    cio = jax.lax.broadcasted_iota(_i32, (N, 1), 0).astype(_f32)
    posT = jnp.where(oc > 0.0, c - 1.0, nob + cio - c)
    rio = jax.lax.broadcasted_iota(_i32, (N, N), 1)
    PT = (rio == posT.astype(_i32)).astype(_f32)
    sel = cio < nob
    yc = jnp.where(sel, ymix_ref[0], ydec_ref[0])
    y_ref[0] = PT @ yc + bh_ref[...]


def kernel(x_full, obs_mask, W_patch, b_patch, Wt1, conv_k, Wt2, var_emb,
           Wq, Wk, Wv, Wo, Wq2, Wk2, Wv2, Wo2, C, W_head, b_head):
    obsf = obs_mask.astype(_f32)
    obs_col = obsf.reshape(B, N, 1)
    obs_row = obsf.reshape(B, 1, N)
    bp2 = b_patch.reshape(1, D)
    ckT = conv_k.T
    bh2 = b_head.reshape(1, PRED)
    CT = C.T
    Whs = W_head.reshape(L, D, PRED).sum(axis=0)

    g2 = lambda b: (0, 0)
    g3 = lambda b: (b, 0, 0)
    posg, nobs = pl.pallas_call(
        _route_body,
        grid=(B,),
        in_specs=[pl.BlockSpec((1, 1, N), g3)],
        out_specs=[
            pl.BlockSpec((1, 1, N), g3),
            pl.BlockSpec((1, 1, 1), g3),
        ],
        out_shape=[
            jax.ShapeDtypeStruct((B, 1, N), _i32),
            jax.ShapeDtypeStruct((B, 1, 1), _f32),
        ],
    )(obs_row)

    xve = jnp.concatenate(
        [x_full.reshape(_ROWS, T), jnp.zeros((_ROWS, 128 - T), _f32),
         jnp.tile(var_emb, (B, 1)), jnp.zeros((_ROWS, 128 - D), _f32)],
        axis=1)
    sc_scatter = functools.partial(
        pl.kernel, _sc_scatter_body,
        mesh=plsc.VectorSubcoreMesh(core_axis_name="c", subcore_axis_name="s"),
        out_type=jax.ShapeDtypeStruct((_ROWS, _CW), _f32),
        scratch_types=[
            pltpu.VMEM((_WPER,), _i32),
            pltpu.VMEM((_WPER, _CW), _f32),
            pltpu.SemaphoreType.DMA,
        ],
    )()
    xveg = sc_scatter(xve, posg.reshape(_ROWS))
    xg = xveg[:, :T].reshape(B, N, T)
    veg = xveg[:, 128:128 + D].reshape(B, N, D)

    x_pad = jnp.pad(xg, ((0, 0), (0, 0), (0, 2)))
    widx = jnp.arange(LP)[:, None] * STRIDE + jnp.arange(PATCH)[None, :]
    patches_tm = x_pad[:, :, widx].transpose(0, 2, 1, 3)

    w2 = lambda b, n: (0, 0)
    nb3 = lambda b, n: (b, 0, 0)
    tok, y_mix = pl.pallas_call(
        _mixer_body,
        grid=(B, NB),
        in_specs=[
            pl.BlockSpec((1, LP, RBLK, PATCH), lambda b, n: (b, 0, n, 0)),
            pl.BlockSpec((1, RBLK, D), lambda b, n: (b, n, 0)),
            pl.BlockSpec((1, 1, 1), nb3),
            pl.BlockSpec((PATCH, D), w2),
            pl.BlockSpec((1, D), w2),
            pl.BlockSpec((D, 2 * D), w2),
            pl.BlockSpec((PATCH, 2 * D), w2),
            pl.BlockSpec((2 * D, D), w2),
            pl.BlockSpec((L * D, PRED), w2),
        ],
        out_specs=[
            pl.BlockSpec((1, RBLK, D), lambda b, n: (b, n, 0)),
            pl.BlockSpec((1, RBLK, PRED), lambda b, n: (b, n, 0)),
        ],
        out_shape=[
            jax.ShapeDtypeStruct((B, N, D), _f32),
            jax.ShapeDtypeStruct((B, N, PRED), _f32),
        ],
    )(patches_tm, veg, nobs, W_patch, bp2, Wt1, ckT, Wt2, W_head)

    for lyr in range(NLAYERS):
        tok = pl.pallas_call(
            _enc_body,
            grid=(B, NB),
            in_specs=[
                pl.BlockSpec((1, N, D), nb3),
                pl.BlockSpec((1, 1, 1), nb3),
                pl.BlockSpec((D, D), w2),
                pl.BlockSpec((D, D), w2),
                pl.BlockSpec((D, D), w2),
                pl.BlockSpec((D, D), w2),
            ],
            out_specs=pl.BlockSpec((1, RBLK, D), lambda b, n: (b, n, 0)),
            out_shape=jax.ShapeDtypeStruct((B, N, D), _f32),
        )(tok, nobs, Wq[lyr], Wk[lyr], Wv[lyr], Wo[lyr])

    y_dec, q_sub, w_sub = pl.pallas_call(
        _dec_body,
        grid=(B, NB),
        in_specs=[
            pl.BlockSpec((1, N, D), nb3),
            pl.BlockSpec((1, RBLK, D), lambda b, n: (b, n, 0)),
            pl.BlockSpec((1, 1, 1), nb3),
            pl.BlockSpec((D, D), w2),
            pl.BlockSpec((D, D), w2),
            pl.BlockSpec((D, D), w2),
            pl.BlockSpec((D, D), w2),
            pl.BlockSpec((K, D), w2),
            pl.BlockSpec((D, K), w2),
            pl.BlockSpec((D, PRED), w2),
        ],
        out_specs=[
            pl.BlockSpec((1, RBLK, PRED), lambda b, n: (b, n, 0)),
            pl.BlockSpec((1, 1, D), nb3),
            pl.BlockSpec((1, 1, K), nb3),
        ],
        out_shape=[
            jax.ShapeDtypeStruct((B, N, PRED), _f32),
            jax.ShapeDtypeStruct((B, 1, D), _f32),
            jax.ShapeDtypeStruct((B, 1, K), _f32),
        ],
    )(tok, veg, nobs, Wq2, Wk2, Wv2, Wo2, C, CT, Whs)

    y_hat = pl.pallas_call(
        _fin_body,
        grid=(B,),
        in_specs=[
            pl.BlockSpec((1, N, 1), g3),
            pl.BlockSpec((1, N, PRED), g3),
            pl.BlockSpec((1, N, PRED), g3),
            pl.BlockSpec((1, PRED), g2),
        ],
        out_specs=pl.BlockSpec((1, N, PRED), g3),
        out_shape=jax.ShapeDtypeStruct((B, N, PRED), _f32),
    )(obs_col, y_mix, y_dec, bh2)

    return (y_hat, q_sub.reshape(B, D), w_sub.reshape(B, K))


# speedup vs baseline: 1.1799x; 1.0227x over previous
import functools
import math

import jax
import jax.numpy as jnp
from jax import lax
from jax.experimental import pallas as pl
from jax.experimental.pallas import tpu as pltpu
from jax.experimental.pallas import tpu_sc as plsc

B, N, T = 4, 1024, 96
D, H, NLAYERS = 64, 8, 2
PATCH, STRIDE = 4, 2
L = (T - PATCH) // STRIDE + 1
LP = 48
K, TAU, PRED = 16, 0.5, 24
DH = D // H
RBLK = 128
NB = N // RBLK

_i32 = jnp.int32
_f32 = jnp.float32


def _route_body(obsr_ref, posg_ref, nobs_ref):
    o = obsr_ref[0]
    c = o
    s = 1
    while s < N:
        c = c + jnp.concatenate(
            [jnp.zeros((1, s), _f32), c[:, :N - s]], axis=1)
        s *= 2
    nob = c[:, N - 1:N]
    iota_r = jax.lax.broadcasted_iota(_i32, (1, N), 1).astype(_f32)
    pos = jnp.where(o > 0.0, c - 1.0, nob + iota_r - c)
    posg_ref[0] = pos.astype(_i32) + pl.program_id(0) * N
    nobs_ref[0] = nob


_SC_NC, _SC_NW = 2, 32
_ROWS = B * N
_CW = 128
_WPER = _ROWS // _SC_NW


def _sc_scatter_body(xp_hbm, vep_hbm, posg_hbm, xgp_hbm, vegp_hbm,
                     idx_v, xrows_v, verows_v, sem1, sem2):
    wid = lax.axis_index("s") * _SC_NC + lax.axis_index("c")
    base = wid * _WPER
    nbase = base - (base // N) * N
    pltpu.sync_copy(posg_hbm.at[pl.ds(base, _WPER)], idx_v)
    pltpu.sync_copy(xp_hbm.at[pl.ds(base, _WPER)], xrows_v)
    pltpu.sync_copy(vep_hbm.at[pl.ds(nbase, _WPER)], verows_v)
    cp1 = pltpu.async_copy(xrows_v, xgp_hbm.at[idx_v], sem1)
    cp2 = pltpu.async_copy(verows_v, vegp_hbm.at[idx_v], sem2)
    cp1.wait()
    cp2.wait()


def _mixer_body(pt_ref, veg_ref, nobs_ref, Wp_ref, bp_ref, Wt1_ref, ck_ref,
                Wt2_ref, Wh_ref, tok_ref, y_ref):
    nob_i = nobs_ref[0, 0, 0].astype(_i32)

    @pl.when(pl.program_id(1) * RBLK < nob_i)
    def _():
        pt = pt_ref[0].reshape(LP * RBLK, PATCH)
        h = pt @ Wp_ref[...] + bp_ref[...]
        u = (h @ Wt1_ref[...]).reshape(LP, RBLK, 2 * D)
        c0 = ck_ref[0:1, :][None]
        c1 = ck_ref[1:2, :][None]
        c2 = ck_ref[2:3, :][None]
        c3 = ck_ref[3:4, :][None]
        v = (u * c3
             + jnp.concatenate([jnp.zeros((1, RBLK, 2 * D), _f32),
                                u[:LP - 1]], axis=0) * c2
             + jnp.concatenate([jnp.zeros((2, RBLK, 2 * D), _f32),
                                u[:LP - 2]], axis=0) * c1
             + jnp.concatenate([jnp.zeros((3, RBLK, 2 * D), _f32),
                                u[:LP - 3]], axis=0) * c0)
        sil = (v * jax.nn.sigmoid(v)).reshape(LP * RBLK, 2 * D)
        h2 = (h + sil @ Wt2_ref[...]).reshape(LP, RBLK, D)
        tok_ref[0] = (jnp.sum(h2[:L], axis=0) * _f32(1.0 / L) + veg_ref[0])
        acc = h2[0] @ Wh_ref[0:D, :]
        for l in range(1, L):
            acc = acc + h2[l] @ Wh_ref[l * D:(l + 1) * D, :]
        y_ref[0] = acc


_KD = (((1,), (1,)), ((), ()))


def _enc_body(tok_ref, nobs_ref, Wq_ref, Wk_ref, Wv_ref, Wo_ref, out_ref):
    nob = nobs_ref[0, 0, 0]
    nob_i = nob.astype(_i32)
    qb = pl.program_id(1)

    @pl.when(qb * RBLK < nob_i)
    def _():
        tokf = tok_ref[0]
        cio = jax.lax.broadcasted_iota(_i32, (N, 1), 0)
        tokm = jnp.where(cio < nob_i, tokf, 0.0)
        tq = tok_ref[0, pl.ds(qb * RBLK, RBLK), :]
        q = tq @ Wq_ref[...]
        kk = tokm @ Wk_ref[...]
        vv = tokm @ Wv_ref[...]
        rio = jax.lax.broadcasted_iota(_i32, (1, N), 1)
        bias = jnp.where(rio < nob_i, 0.0, -1e9).astype(_f32)
        s1 = _f32(1.0 / math.sqrt(float(DH)))
        outs = []
        for hh in range(H):
            qh = q[:, hh * DH:(hh + 1) * DH]
            kh = kk[:, hh * DH:(hh + 1) * DH]
            vh = vv[:, hh * DH:(hh + 1) * DH]
            sc = jax.lax.dot_general(qh, kh, _KD) * s1 + bias
            m = jnp.max(sc, axis=-1, keepdims=True)
            e = jnp.exp(sc - m)
            ssum = jnp.sum(e, axis=-1, keepdims=True)
            outs.append((e @ vh) / ssum)
        out_ref[0] = tq + jnp.concatenate(outs, axis=1) @ Wo_ref[...]


def _dec_body(tok_ref, veg_ref, nobs_ref, Wq2_ref, Wk2_ref, Wv2_ref, Wo2_ref,
              C_ref, CT_ref, Whs_ref, ydec_ref, qsub_ref, wsub_ref):
    nob = nobs_ref[0, 0, 0]
    nob_i = nob.astype(_i32)
    qb = pl.program_id(1)
    tokf = tok_ref[0]
    cio = jax.lax.broadcasted_iota(_i32, (N, 1), 0)
    tokm = jnp.where(cio < nob_i, tokf, 0.0)
    qsub = jnp.sum(tokm, axis=0, keepdims=True) / nob
    CT = CT_ref[...]
    cn2 = jnp.sum(CT * CT, axis=0, keepdims=True)
    logits = (2.0 * (qsub @ CT) - cn2) * _f32(1.0 / TAU)
    m = jnp.max(logits, axis=-1, keepdims=True)
    e = jnp.exp(logits - m)
    wsub = e / jnp.sum(e, axis=-1, keepdims=True)

    @pl.when(qb == NB - 1)
    def _():
        qsub_ref[0] = qsub
        wsub_ref[0] = wsub

    @pl.when((qb + 1) * RBLK > nob_i)
    def _():
        mt = veg_ref[0] + (wsub @ C_ref[...])
        q2 = mt @ Wq2_ref[...]
        k2 = tokm @ Wk2_ref[...]
        v2 = tokm @ Wv2_ref[...]
        rio = jax.lax.broadcasted_iota(_i32, (1, N), 1)
        bias = jnp.where(rio < nob_i, 0.0, -1e9).astype(_f32)
        sc2 = (jax.lax.dot_general(q2, k2, _KD) * _f32(1.0 / math.sqrt(float(D)))
               + bias)
        m2 = jnp.max(sc2, axis=-1, keepdims=True)
        e2 = jnp.exp(sc2 - m2)
        s2 = jnp.sum(e2, axis=-1, keepdims=True)
        mo = mt + ((e2 @ v2) / s2) @ Wo2_ref[...]
        ydec_ref[0] = mo @ Whs_ref[...]


def _fin_body(obsc_ref, ymix_ref, ydec_ref, bh_ref, y_ref):
    oc = obsc_ref[0]
    c = oc
    s = 1
    while s < N:
        c = c + jnp.concatenate(
            [jnp.zeros((s, 1), _f32), c[:N - s]], axis=0)
        s *= 2
    nob = c[N - 1:N, :]
    cio = jax.lax.broadcasted_iota(_i32, (N, 1), 0).astype(_f32)
    posT = jnp.where(oc > 0.0, c - 1.0, nob + cio - c)
    rio = jax.lax.broadcasted_iota(_i32, (N, N), 1)
    PT = (rio == posT.astype(_i32)).astype(_f32)
    sel = cio < nob
    yc = jnp.where(sel, ymix_ref[0], ydec_ref[0])
    y_ref[0] = PT @ yc + bh_ref[...]


def kernel(x_full, obs_mask, W_patch, b_patch, Wt1, conv_k, Wt2, var_emb,
           Wq, Wk, Wv, Wo, Wq2, Wk2, Wv2, Wo2, C, W_head, b_head):
    obsf = obs_mask.astype(_f32)
    obs_col = obsf.reshape(B, N, 1)
    obs_row = obsf.reshape(B, 1, N)
    bp2 = b_patch.reshape(1, D)
    ckT = conv_k.T
    bh2 = b_head.reshape(1, PRED)
    CT = C.T
    Whs = W_head.reshape(L, D, PRED).sum(axis=0)

    g2 = lambda b: (0, 0)
    g3 = lambda b: (b, 0, 0)
    posg, nobs = pl.pallas_call(
        _route_body,
        grid=(B,),
        in_specs=[pl.BlockSpec((1, 1, N), g3)],
        out_specs=[
            pl.BlockSpec((1, 1, N), g3),
            pl.BlockSpec((1, 1, 1), g3),
        ],
        out_shape=[
            jax.ShapeDtypeStruct((B, 1, N), _i32),
            jax.ShapeDtypeStruct((B, 1, 1), _f32),
        ],
    )(obs_row)

    xp = jnp.pad(x_full.reshape(_ROWS, T), ((0, 0), (0, 128 - T)))
    vep = jnp.pad(var_emb, ((0, 0), (0, 128 - D)))
    sc_scatter = functools.partial(
        pl.kernel, _sc_scatter_body,
        mesh=plsc.VectorSubcoreMesh(core_axis_name="c", subcore_axis_name="s"),
        out_type=[
            jax.ShapeDtypeStruct((_ROWS, 128), _f32),
            jax.ShapeDtypeStruct((_ROWS, 128), _f32),
        ],
        scratch_types=[
            pltpu.VMEM((_WPER,), _i32),
            pltpu.VMEM((_WPER, 128), _f32),
            pltpu.VMEM((_WPER, 128), _f32),
            pltpu.SemaphoreType.DMA,
            pltpu.SemaphoreType.DMA,
        ],
    )()
    xgp, vegp = sc_scatter(xp, vep, posg.reshape(_ROWS))
    xg = xgp[:, :T].reshape(B, N, T)
    veg = vegp[:, :D].reshape(B, N, D)

    x_pad = jnp.pad(xg, ((0, 0), (0, 0), (0, 2)))
    widx = jnp.arange(LP)[:, None] * STRIDE + jnp.arange(PATCH)[None, :]
    patches_tm = x_pad[:, :, widx].transpose(0, 2, 1, 3)

    w2 = lambda b, n: (0, 0)
    nb3 = lambda b, n: (b, 0, 0)
    tok, y_mix = pl.pallas_call(
        _mixer_body,
        grid=(B, NB),
        in_specs=[
            pl.BlockSpec((1, LP, RBLK, PATCH), lambda b, n: (b, 0, n, 0)),
            pl.BlockSpec((1, RBLK, D), lambda b, n: (b, n, 0)),
            pl.BlockSpec((1, 1, 1), nb3),
            pl.BlockSpec((PATCH, D), w2),
            pl.BlockSpec((1, D), w2),
            pl.BlockSpec((D, 2 * D), w2),
            pl.BlockSpec((PATCH, 2 * D), w2),
            pl.BlockSpec((2 * D, D), w2),
            pl.BlockSpec((L * D, PRED), w2),
        ],
        out_specs=[
            pl.BlockSpec((1, RBLK, D), lambda b, n: (b, n, 0)),
            pl.BlockSpec((1, RBLK, PRED), lambda b, n: (b, n, 0)),
        ],
        out_shape=[
            jax.ShapeDtypeStruct((B, N, D), _f32),
            jax.ShapeDtypeStruct((B, N, PRED), _f32),
        ],
    )(patches_tm, veg, nobs, W_patch, bp2, Wt1, ckT, Wt2, W_head)

    for lyr in range(NLAYERS):
        tok = pl.pallas_call(
            _enc_body,
            grid=(B, NB),
            in_specs=[
                pl.BlockSpec((1, N, D), nb3),
                pl.BlockSpec((1, 1, 1), nb3),
                pl.BlockSpec((D, D), w2),
                pl.BlockSpec((D, D), w2),
                pl.BlockSpec((D, D), w2),
                pl.BlockSpec((D, D), w2),
            ],
            out_specs=pl.BlockSpec((1, RBLK, D), lambda b, n: (b, n, 0)),
            out_shape=jax.ShapeDtypeStruct((B, N, D), _f32),
        )(tok, nobs, Wq[lyr], Wk[lyr], Wv[lyr], Wo[lyr])

    y_dec, q_sub, w_sub = pl.pallas_call(
        _dec_body,
        grid=(B, NB),
        in_specs=[
            pl.BlockSpec((1, N, D), nb3),
            pl.BlockSpec((1, RBLK, D), lambda b, n: (b, n, 0)),
            pl.BlockSpec((1, 1, 1), nb3),
            pl.BlockSpec((D, D), w2),
            pl.BlockSpec((D, D), w2),
            pl.BlockSpec((D, D), w2),
            pl.BlockSpec((D, D), w2),
            pl.BlockSpec((K, D), w2),
            pl.BlockSpec((D, K), w2),
            pl.BlockSpec((D, PRED), w2),
        ],
        out_specs=[
            pl.BlockSpec((1, RBLK, PRED), lambda b, n: (b, n, 0)),
            pl.BlockSpec((1, 1, D), nb3),
            pl.BlockSpec((1, 1, K), nb3),
        ],
        out_shape=[
            jax.ShapeDtypeStruct((B, N, PRED), _f32),
            jax.ShapeDtypeStruct((B, 1, D), _f32),
            jax.ShapeDtypeStruct((B, 1, K), _f32),
        ],
    )(tok, veg, nobs, Wq2, Wk2, Wv2, Wo2, C, CT, Whs)

    y_hat = pl.pallas_call(
        _fin_body,
        grid=(B,),
        in_specs=[
            pl.BlockSpec((1, N, 1), g3),
            pl.BlockSpec((1, N, PRED), g3),
            pl.BlockSpec((1, N, PRED), g3),
            pl.BlockSpec((1, PRED), g2),
        ],
        out_specs=pl.BlockSpec((1, N, PRED), g3),
        out_shape=jax.ShapeDtypeStruct((B, N, PRED), _f32),
    )(obs_col, y_mix, y_dec, bh2)

    return (y_hat, q_sub.reshape(B, D), w_sub.reshape(B, K))


# speedup vs baseline: 1.2213x; 1.0351x over previous
import functools
import math

import jax
import jax.numpy as jnp
from jax import lax
from jax.experimental import pallas as pl
from jax.experimental.pallas import tpu as pltpu
from jax.experimental.pallas import tpu_sc as plsc

B, N, T = 4, 1024, 96
D, H, NLAYERS = 64, 8, 2
PATCH, STRIDE = 4, 2
L = (T - PATCH) // STRIDE + 1
LP = 48
K, TAU, PRED = 16, 0.5, 24
DH = D // H
RBLK = 128
NB = N // RBLK

_i32 = jnp.int32
_f32 = jnp.float32


def _route_body(obsr_ref, posg_ref, nobs_ref):
    o = obsr_ref[0]
    c = o
    s = 1
    while s < N:
        c = c + jnp.concatenate(
            [jnp.zeros((1, s), _f32), c[:, :N - s]], axis=1)
        s *= 2
    nob = c[:, N - 1:N]
    iota_r = jax.lax.broadcasted_iota(_i32, (1, N), 1).astype(_f32)
    pos = jnp.where(o > 0.0, c - 1.0, nob + iota_r - c)
    posg_ref[0] = pos.astype(_i32) + pl.program_id(0) * N
    nobs_ref[0] = nob


_SC_NC, _SC_NW = 2, 32
_ROWS = B * N
_CW = 128
_WPER = _ROWS // _SC_NW


def _sc_scatter_body(xp_hbm, vep_hbm, posg_hbm, xgp_hbm, vegp_hbm,
                     idx_v, xrows_v, verows_v, sem1, sem2):
    wid = lax.axis_index("s") * _SC_NC + lax.axis_index("c")
    base = wid * _WPER
    nbase = base - (base // N) * N
    pltpu.sync_copy(posg_hbm.at[pl.ds(base, _WPER)], idx_v)
    pltpu.sync_copy(xp_hbm.at[pl.ds(base, _WPER)], xrows_v)
    pltpu.sync_copy(vep_hbm.at[pl.ds(nbase, _WPER)], verows_v)
    cp1 = pltpu.async_copy(xrows_v, xgp_hbm.at[idx_v], sem1)
    cp2 = pltpu.async_copy(verows_v, vegp_hbm.at[idx_v], sem2)
    cp1.wait()
    cp2.wait()


def _mixer_body(pt_ref, veg_ref, nobs_ref, Wp_ref, bp_ref, Wt1_ref, ck_ref,
                Wt2_ref, Wh_ref, tok_ref, y_ref):
    nob_i = nobs_ref[0, 0, 0].astype(_i32)

    @pl.when(pl.program_id(1) * RBLK < nob_i)
    def _():
        pt = pt_ref[0].reshape(LP * RBLK, PATCH)
        h = pt @ Wp_ref[...] + bp_ref[...]
        u = (h @ Wt1_ref[...]).reshape(LP, RBLK, 2 * D)
        c0 = ck_ref[0:1, :][None]
        c1 = ck_ref[1:2, :][None]
        c2 = ck_ref[2:3, :][None]
        c3 = ck_ref[3:4, :][None]
        v = (u * c3
             + jnp.concatenate([jnp.zeros((1, RBLK, 2 * D), _f32),
                                u[:LP - 1]], axis=0) * c2
             + jnp.concatenate([jnp.zeros((2, RBLK, 2 * D), _f32),
                                u[:LP - 2]], axis=0) * c1
             + jnp.concatenate([jnp.zeros((3, RBLK, 2 * D), _f32),
                                u[:LP - 3]], axis=0) * c0)
        sil = (v * jax.nn.sigmoid(v)).reshape(LP * RBLK, 2 * D)
        h2 = (h + sil @ Wt2_ref[...]).reshape(LP, RBLK, D)
        tok_ref[0] = (jnp.sum(h2[:L], axis=0) * _f32(1.0 / L) + veg_ref[0])
        acc = h2[0] @ Wh_ref[0:D, :]
        for l in range(1, L):
            acc = acc + h2[l] @ Wh_ref[l * D:(l + 1) * D, :]
        y_ref[0] = acc


_KD = (((1,), (1,)), ((), ()))


_WIDTHS = (256, 512, 768, 1024)


def _enc_attn(tok_ref, nob_i, tq, Wq_ref, Wk_ref, Wv_ref, Wo_ref, W):
    tokf = tok_ref[0, pl.ds(0, W), :]
    cio = jax.lax.broadcasted_iota(_i32, (W, 1), 0)
    tokm = jnp.where(cio < nob_i, tokf, 0.0)
    q = tq @ Wq_ref[...]
    kk = tokm @ Wk_ref[...]
    vv = tokm @ Wv_ref[...]
    rio = jax.lax.broadcasted_iota(_i32, (1, W), 1)
    bias = jnp.where(rio < nob_i, 0.0, -1e9).astype(_f32)
    s1 = _f32(1.0 / math.sqrt(float(DH)))
    outs = []
    for hh in range(H):
        qh = q[:, hh * DH:(hh + 1) * DH]
        kh = kk[:, hh * DH:(hh + 1) * DH]
        vh = vv[:, hh * DH:(hh + 1) * DH]
        sc = jax.lax.dot_general(qh, kh, _KD) * s1 + bias
        m = jnp.max(sc, axis=-1, keepdims=True)
        e = jnp.exp(sc - m)
        ssum = jnp.sum(e, axis=-1, keepdims=True)
        outs.append((e @ vh) / ssum)
    return tq + jnp.concatenate(outs, axis=1) @ Wo_ref[...]


def _enc_body(tok_ref, nobs_ref, Wq_ref, Wk_ref, Wv_ref, Wo_ref, out_ref):
    nob = nobs_ref[0, 0, 0]
    nob_i = nob.astype(_i32)
    qb = pl.program_id(1)

    @pl.when(qb * RBLK < nob_i)
    def _():
        tq = tok_ref[0, pl.ds(qb * RBLK, RBLK), :]
        for wi, W in enumerate(_WIDTHS):
            lo = 0 if wi == 0 else _WIDTHS[wi - 1]

            @pl.when((nob_i > lo) & (nob_i <= W))
            def _(W=W):
                out_ref[0] = _enc_attn(tok_ref, nob_i, tq, Wq_ref, Wk_ref,
                                       Wv_ref, Wo_ref, W)


def _dec_attn(tok_ref, nob_i, mt, Wq2_ref, Wk2_ref, Wv2_ref, Wo2_ref,
              Whs_ref, W):
    q2 = mt @ Wq2_ref[...]
    tokf = tok_ref[0, pl.ds(0, W), :]
    cio = jax.lax.broadcasted_iota(_i32, (W, 1), 0)
    tokm = jnp.where(cio < nob_i, tokf, 0.0)
    k2 = tokm @ Wk2_ref[...]
    v2 = tokm @ Wv2_ref[...]
    rio = jax.lax.broadcasted_iota(_i32, (1, W), 1)
    bias = jnp.where(rio < nob_i, 0.0, -1e9).astype(_f32)
    sc2 = (jax.lax.dot_general(q2, k2, _KD)
           * _f32(1.0 / math.sqrt(float(D))) + bias)
    m2 = jnp.max(sc2, axis=-1, keepdims=True)
    e2 = jnp.exp(sc2 - m2)
    s2 = jnp.sum(e2, axis=-1, keepdims=True)
    mo = mt + ((e2 @ v2) / s2) @ Wo2_ref[...]
    return mo @ Whs_ref[...]


def _dec_body(tok_ref, veg_ref, nobs_ref, Wq2_ref, Wk2_ref, Wv2_ref, Wo2_ref,
              C_ref, CT_ref, Whs_ref, ydec_ref, qsub_ref, wsub_ref):
    nob = nobs_ref[0, 0, 0]
    nob_i = nob.astype(_i32)
    qb = pl.program_id(1)
    tokf = tok_ref[0]
    cio = jax.lax.broadcasted_iota(_i32, (N, 1), 0)
    tokm = jnp.where(cio < nob_i, tokf, 0.0)
    qsub = jnp.sum(tokm, axis=0, keepdims=True) / nob
    CT = CT_ref[...]
    cn2 = jnp.sum(CT * CT, axis=0, keepdims=True)
    logits = (2.0 * (qsub @ CT) - cn2) * _f32(1.0 / TAU)
    m = jnp.max(logits, axis=-1, keepdims=True)
    e = jnp.exp(logits - m)
    wsub = e / jnp.sum(e, axis=-1, keepdims=True)

    @pl.when(qb == NB - 1)
    def _():
        qsub_ref[0] = qsub
        wsub_ref[0] = wsub

    @pl.when((qb + 1) * RBLK > nob_i)
    def _():
        mt = veg_ref[0] + (wsub @ C_ref[...])
        for wi, W in enumerate(_WIDTHS):
            lo = 0 if wi == 0 else _WIDTHS[wi - 1]

            @pl.when((nob_i > lo) & (nob_i <= W))
            def _(W=W):
                ydec_ref[0] = _dec_attn(tok_ref, nob_i, mt, Wq2_ref,
                                        Wk2_ref, Wv2_ref, Wo2_ref,
                                        Whs_ref, W)


def _fin_body(obsc_ref, ymix_ref, ydec_ref, bh_ref, y_ref):
    oc = obsc_ref[0]
    c = oc
    s = 1
    while s < N:
        c = c + jnp.concatenate(
            [jnp.zeros((s, 1), _f32), c[:N - s]], axis=0)
        s *= 2
    nob = c[N - 1:N, :]
    cio = jax.lax.broadcasted_iota(_i32, (N, 1), 0).astype(_f32)
    posT = jnp.where(oc > 0.0, c - 1.0, nob + cio - c)
    rio = jax.lax.broadcasted_iota(_i32, (N, N), 1)
    PT = (rio == posT.astype(_i32)).astype(_f32)
    sel = cio < nob
    yc = jnp.where(sel, ymix_ref[0], ydec_ref[0])
    y_ref[0] = PT @ yc + bh_ref[...]


def kernel(x_full, obs_mask, W_patch, b_patch, Wt1, conv_k, Wt2, var_emb,
           Wq, Wk, Wv, Wo, Wq2, Wk2, Wv2, Wo2, C, W_head, b_head):
    obsf = obs_mask.astype(_f32)
    obs_col = obsf.reshape(B, N, 1)
    obs_row = obsf.reshape(B, 1, N)
    bp2 = b_patch.reshape(1, D)
    ckT = conv_k.T
    bh2 = b_head.reshape(1, PRED)
    CT = C.T
    Whs = W_head.reshape(L, D, PRED).sum(axis=0)

    g2 = lambda b: (0, 0)
    g3 = lambda b: (b, 0, 0)
    posg, nobs = pl.pallas_call(
        _route_body,
        grid=(B,),
        in_specs=[pl.BlockSpec((1, 1, N), g3)],
        out_specs=[
            pl.BlockSpec((1, 1, N), g3),
            pl.BlockSpec((1, 1, 1), g3),
        ],
        out_shape=[
            jax.ShapeDtypeStruct((B, 1, N), _i32),
            jax.ShapeDtypeStruct((B, 1, 1), _f32),
        ],
    )(obs_row)

    xp = jnp.pad(x_full.reshape(_ROWS, T), ((0, 0), (0, 128 - T)))
    vep = jnp.pad(var_emb, ((0, 0), (0, 128 - D)))
    sc_scatter = functools.partial(
        pl.kernel, _sc_scatter_body,
        mesh=plsc.VectorSubcoreMesh(core_axis_name="c", subcore_axis_name="s"),
        out_type=[
            jax.ShapeDtypeStruct((_ROWS, 128), _f32),
            jax.ShapeDtypeStruct((_ROWS, 128), _f32),
        ],
        scratch_types=[
            pltpu.VMEM((_WPER,), _i32),
            pltpu.VMEM((_WPER, 128), _f32),
            pltpu.VMEM((_WPER, 128), _f32),
            pltpu.SemaphoreType.DMA,
            pltpu.SemaphoreType.DMA,
        ],
    )()
    xgp, vegp = sc_scatter(xp, vep, posg.reshape(_ROWS))
    xg = xgp[:, :T].reshape(B, N, T)
    veg = vegp[:, :D].reshape(B, N, D)

    x_pad = jnp.pad(xg, ((0, 0), (0, 0), (0, 2)))
    widx = jnp.arange(LP)[:, None] * STRIDE + jnp.arange(PATCH)[None, :]
    patches_tm = x_pad[:, :, widx].transpose(0, 2, 1, 3)

    w2 = lambda b, n: (0, 0)
    nb3 = lambda b, n: (b, 0, 0)
    tok, y_mix = pl.pallas_call(
        _mixer_body,
        grid=(B, NB),
        in_specs=[
            pl.BlockSpec((1, LP, RBLK, PATCH), lambda b, n: (b, 0, n, 0)),
            pl.BlockSpec((1, RBLK, D), lambda b, n: (b, n, 0)),
            pl.BlockSpec((1, 1, 1), nb3),
            pl.BlockSpec((PATCH, D), w2),
            pl.BlockSpec((1, D), w2),
            pl.BlockSpec((D, 2 * D), w2),
            pl.BlockSpec((PATCH, 2 * D), w2),
            pl.BlockSpec((2 * D, D), w2),
            pl.BlockSpec((L * D, PRED), w2),
        ],
        out_specs=[
            pl.BlockSpec((1, RBLK, D), lambda b, n: (b, n, 0)),
            pl.BlockSpec((1, RBLK, PRED), lambda b, n: (b, n, 0)),
        ],
        out_shape=[
            jax.ShapeDtypeStruct((B, N, D), _f32),
            jax.ShapeDtypeStruct((B, N, PRED), _f32),
        ],
    )(patches_tm, veg, nobs, W_patch, bp2, Wt1, ckT, Wt2, W_head)

    for lyr in range(NLAYERS):
        tok = pl.pallas_call(
            _enc_body,
            grid=(B, NB),
            in_specs=[
                pl.BlockSpec((1, N, D), nb3),
                pl.BlockSpec((1, 1, 1), nb3),
                pl.BlockSpec((D, D), w2),
                pl.BlockSpec((D, D), w2),
                pl.BlockSpec((D, D), w2),
                pl.BlockSpec((D, D), w2),
            ],
            out_specs=pl.BlockSpec((1, RBLK, D), lambda b, n: (b, n, 0)),
            out_shape=jax.ShapeDtypeStruct((B, N, D), _f32),
        )(tok, nobs, Wq[lyr], Wk[lyr], Wv[lyr], Wo[lyr])

    y_dec, q_sub, w_sub = pl.pallas_call(
        _dec_body,
        grid=(B, NB),
        in_specs=[
            pl.BlockSpec((1, N, D), nb3),
            pl.BlockSpec((1, RBLK, D), lambda b, n: (b, n, 0)),
            pl.BlockSpec((1, 1, 1), nb3),
            pl.BlockSpec((D, D), w2),
            pl.BlockSpec((D, D), w2),
            pl.BlockSpec((D, D), w2),
            pl.BlockSpec((D, D), w2),
            pl.BlockSpec((K, D), w2),
            pl.BlockSpec((D, K), w2),
            pl.BlockSpec((D, PRED), w2),
        ],
        out_specs=[
            pl.BlockSpec((1, RBLK, PRED), lambda b, n: (b, n, 0)),
            pl.BlockSpec((1, 1, D), nb3),
            pl.BlockSpec((1, 1, K), nb3),
        ],
        out_shape=[
            jax.ShapeDtypeStruct((B, N, PRED), _f32),
            jax.ShapeDtypeStruct((B, 1, D), _f32),
            jax.ShapeDtypeStruct((B, 1, K), _f32),
        ],
    )(tok, veg, nobs, Wq2, Wk2, Wv2, Wo2, C, CT, Whs)

    y_hat = pl.pallas_call(
        _fin_body,
        grid=(B,),
        in_specs=[
            pl.BlockSpec((1, N, 1), g3),
            pl.BlockSpec((1, N, PRED), g3),
            pl.BlockSpec((1, N, PRED), g3),
            pl.BlockSpec((1, PRED), g2),
        ],
        out_specs=pl.BlockSpec((1, N, PRED), g3),
        out_shape=jax.ShapeDtypeStruct((B, N, PRED), _f32),
    )(obs_col, y_mix, y_dec, bh2)

    return (y_hat, q_sub.reshape(B, D), w_sub.reshape(B, K))


# speedup vs baseline: 1.2311x; 1.0081x over previous
import functools
import math

import jax
import jax.numpy as jnp
from jax import lax
from jax.experimental import pallas as pl
from jax.experimental.pallas import tpu as pltpu
from jax.experimental.pallas import tpu_sc as plsc

B, N, T = 4, 1024, 96
D, H, NLAYERS = 64, 8, 2
PATCH, STRIDE = 4, 2
L = (T - PATCH) // STRIDE + 1
LP = 48
K, TAU, PRED = 16, 0.5, 24
DH = D // H
RBLK = 128
NB = N // RBLK

_i32 = jnp.int32
_f32 = jnp.float32
_bf16 = jnp.bfloat16
_MM = (((1,), (0,)), ((), ()))


def _bmm(a, b):
    return jax.lax.dot_general(a.astype(_bf16), b.astype(_bf16), _MM,
                               preferred_element_type=_f32)


def _route_body(obsr_ref, posg_ref, nobs_ref):
    o = obsr_ref[0]
    c = o
    s = 1
    while s < N:
        c = c + jnp.concatenate(
            [jnp.zeros((1, s), _f32), c[:, :N - s]], axis=1)
        s *= 2
    nob = c[:, N - 1:N]
    iota_r = jax.lax.broadcasted_iota(_i32, (1, N), 1).astype(_f32)
    pos = jnp.where(o > 0.0, c - 1.0, nob + iota_r - c)
    posg_ref[0] = pos.astype(_i32) + pl.program_id(0) * N
    nobs_ref[0] = nob


_SC_NC, _SC_NW = 2, 32
_ROWS = B * N
_CW = 128
_WPER = _ROWS // _SC_NW


def _sc_scatter_body(xp_hbm, vep_hbm, posg_hbm, xgp_hbm, vegp_hbm,
                     idx_v, xrows_v, verows_v, sem1, sem2):
    wid = lax.axis_index("s") * _SC_NC + lax.axis_index("c")
    base = wid * _WPER
    nbase = base - (base // N) * N
    pltpu.sync_copy(posg_hbm.at[pl.ds(base, _WPER)], idx_v)
    pltpu.sync_copy(xp_hbm.at[pl.ds(base, _WPER)], xrows_v)
    pltpu.sync_copy(vep_hbm.at[pl.ds(nbase, _WPER)], verows_v)
    cp1 = pltpu.async_copy(xrows_v, xgp_hbm.at[idx_v], sem1)
    cp2 = pltpu.async_copy(verows_v, vegp_hbm.at[idx_v], sem2)
    cp1.wait()
    cp2.wait()


def _mixer_body(pt_ref, veg_ref, nobs_ref, Wp_ref, bp_ref, Wt1_ref, ck_ref,
                Wt2_ref, Wh_ref, tok_ref, y_ref):
    nob_i = nobs_ref[0, 0, 0].astype(_i32)

    @pl.when(pl.program_id(1) * RBLK < nob_i)
    def _():
        pt = pt_ref[0].reshape(LP * RBLK, PATCH)
        h = pt @ Wp_ref[...] + bp_ref[...]
        u = _bmm(h, Wt1_ref[...]).reshape(LP, RBLK, 2 * D)
        c0 = ck_ref[0:1, :][None]
        c1 = ck_ref[1:2, :][None]
        c2 = ck_ref[2:3, :][None]
        c3 = ck_ref[3:4, :][None]
        v = (u * c3
             + jnp.concatenate([jnp.zeros((1, RBLK, 2 * D), _f32),
                                u[:LP - 1]], axis=0) * c2
             + jnp.concatenate([jnp.zeros((2, RBLK, 2 * D), _f32),
                                u[:LP - 2]], axis=0) * c1
             + jnp.concatenate([jnp.zeros((3, RBLK, 2 * D), _f32),
                                u[:LP - 3]], axis=0) * c0)
        sil = (v * jax.nn.sigmoid(v)).reshape(LP * RBLK, 2 * D)
        h2 = (h + _bmm(sil, Wt2_ref[...])).reshape(LP, RBLK, D)
        tok_ref[0] = (jnp.sum(h2[:L], axis=0) * _f32(1.0 / L) + veg_ref[0])
        h2b = h2.astype(_bf16)
        Whb = Wh_ref[...].astype(_bf16)
        acc = jax.lax.dot_general(h2b[0], Whb[0:D, :], _MM,
                                  preferred_element_type=_f32)
        for l in range(1, L):
            acc = acc + jax.lax.dot_general(h2b[l], Whb[l * D:(l + 1) * D, :],
                                            _MM, preferred_element_type=_f32)
        y_ref[0] = acc


_KD = (((1,), (1,)), ((), ()))


_WIDTHS = (256, 512, 768, 1024)


def _enc_attn(tok_ref, nob_i, tq, Wq_ref, Wk_ref, Wv_ref, Wo_ref, W):
    tokf = tok_ref[0, pl.ds(0, W), :]
    cio = jax.lax.broadcasted_iota(_i32, (W, 1), 0)
    tokm = jnp.where(cio < nob_i, tokf, 0.0)
    q = tq @ Wq_ref[...]
    kk = tokm @ Wk_ref[...]
    vv = tokm @ Wv_ref[...]
    rio = jax.lax.broadcasted_iota(_i32, (1, W), 1)
    bias = jnp.where(rio < nob_i, 0.0, -1e9).astype(_f32)
    s1 = _f32(1.0 / math.sqrt(float(DH)))
    outs = []
    for hh in range(H):
        qh = q[:, hh * DH:(hh + 1) * DH]
        kh = kk[:, hh * DH:(hh + 1) * DH]
        vh = vv[:, hh * DH:(hh + 1) * DH]
        sc = jax.lax.dot_general(qh, kh, _KD) * s1 + bias
        m = jnp.max(sc, axis=-1, keepdims=True)
        e = jnp.exp(sc - m)
        ssum = jnp.sum(e, axis=-1, keepdims=True)
        outs.append((e @ vh) / ssum)
    return tq + jnp.concatenate(outs, axis=1) @ Wo_ref[...]


def _enc_body(tok_ref, nobs_ref, Wq_ref, Wk_ref, Wv_ref, Wo_ref, out_ref):
    nob = nobs_ref[0, 0, 0]
    nob_i = nob.astype(_i32)
    qb = pl.program_id(1)

    @pl.when(qb * RBLK < nob_i)
    def _():
        tq = tok_ref[0, pl.ds(qb * RBLK, RBLK), :]
        for wi, W in enumerate(_WIDTHS):
            lo = 0 if wi == 0 else _WIDTHS[wi - 1]

            @pl.when((nob_i > lo) & (nob_i <= W))
            def _(W=W):
                out_ref[0] = _enc_attn(tok_ref, nob_i, tq, Wq_ref, Wk_ref,
                                       Wv_ref, Wo_ref, W)


def _dec_attn(tokm, nob_i, mt, Wq2_ref, Wk2_ref, Wv2_ref, Wo2_ref,
              Whs_ref, W):
    q2 = mt @ Wq2_ref[...]
    k2 = tokm @ Wk2_ref[...]
    v2 = tokm @ Wv2_ref[...]
    rio = jax.lax.broadcasted_iota(_i32, (1, W), 1)
    bias = jnp.where(rio < nob_i, 0.0, -1e9).astype(_f32)
    sc2 = (jax.lax.dot_general(q2, k2, _KD)
           * _f32(1.0 / math.sqrt(float(D))) + bias)
    m2 = jnp.max(sc2, axis=-1, keepdims=True)
    e2 = jnp.exp(sc2 - m2)
    s2 = jnp.sum(e2, axis=-1, keepdims=True)
    mo = mt + ((e2 @ v2) / s2) @ Wo2_ref[...]
    return mo @ Whs_ref[...]


def _dec_body(tok_ref, veg_ref, nobs_ref, Wq2_ref, Wk2_ref, Wv2_ref, Wo2_ref,
              C_ref, CT_ref, Whs_ref, ydec_ref, qsub_ref, wsub_ref):
    nob = nobs_ref[0, 0, 0]
    nob_i = nob.astype(_i32)
    qb = pl.program_id(1)

    @pl.when((qb + 1) * RBLK > nob_i)
    def _():
        for wi, W in enumerate(_WIDTHS):
            lo = 0 if wi == 0 else _WIDTHS[wi - 1]

            @pl.when((nob_i > lo) & (nob_i <= W))
            def _(W=W):
                tokf = tok_ref[0, pl.ds(0, W), :]
                cio = jax.lax.broadcasted_iota(_i32, (W, 1), 0)
                tokm = jnp.where(cio < nob_i, tokf, 0.0)
                qsub = jnp.sum(tokm, axis=0, keepdims=True) / nob
                CT = CT_ref[...]
                cn2 = jnp.sum(CT * CT, axis=0, keepdims=True)
                logits = (2.0 * (qsub @ CT) - cn2) * _f32(1.0 / TAU)
                m = jnp.max(logits, axis=-1, keepdims=True)
                e = jnp.exp(logits - m)
                wsub = e / jnp.sum(e, axis=-1, keepdims=True)

                @pl.when(qb == NB - 1)
                def _():
                    qsub_ref[0] = qsub
                    wsub_ref[0] = wsub

                mt = veg_ref[0] + (wsub @ C_ref[...])
                ydec_ref[0] = _dec_attn(tokm, nob_i, mt, Wq2_ref,
                                        Wk2_ref, Wv2_ref, Wo2_ref,
                                        Whs_ref, W)


def _fin_body(obsc_ref, ymix_ref, ydec_ref, bh_ref, y_ref):
    oc = obsc_ref[0]
    c = oc
    s = 1
    while s < N:
        c = c + jnp.concatenate(
            [jnp.zeros((s, 1), _f32), c[:N - s]], axis=0)
        s *= 2
    nob = c[N - 1:N, :]
    cio = jax.lax.broadcasted_iota(_i32, (N, 1), 0).astype(_f32)
    posT = jnp.where(oc > 0.0, c - 1.0, nob + cio - c)
    rio = jax.lax.broadcasted_iota(_i32, (N, N), 1)
    PT = (rio == posT.astype(_i32)).astype(_f32)
    sel = cio < nob
    yc = jnp.where(sel, ymix_ref[0], ydec_ref[0])
    y_ref[0] = PT @ yc + bh_ref[...]


def kernel(x_full, obs_mask, W_patch, b_patch, Wt1, conv_k, Wt2, var_emb,
           Wq, Wk, Wv, Wo, Wq2, Wk2, Wv2, Wo2, C, W_head, b_head):
    obsf = obs_mask.astype(_f32)
    obs_col = obsf.reshape(B, N, 1)
    obs_row = obsf.reshape(B, 1, N)
    bp2 = b_patch.reshape(1, D)
    ckT = conv_k.T
    bh2 = b_head.reshape(1, PRED)
    CT = C.T
    Whs = W_head.reshape(L, D, PRED).sum(axis=0)

    g2 = lambda b: (0, 0)
    g3 = lambda b: (b, 0, 0)
    posg, nobs = pl.pallas_call(
        _route_body,
        grid=(B,),
        in_specs=[pl.BlockSpec((1, 1, N), g3)],
        out_specs=[
            pl.BlockSpec((1, 1, N), g3),
            pl.BlockSpec((1, 1, 1), g3),
        ],
        out_shape=[
            jax.ShapeDtypeStruct((B, 1, N), _i32),
            jax.ShapeDtypeStruct((B, 1, 1), _f32),
        ],
    )(obs_row)

    xp = jnp.pad(x_full.reshape(_ROWS, T), ((0, 0), (0, 128 - T)))
    vep = jnp.pad(var_emb, ((0, 0), (0, 128 - D)))
    sc_scatter = functools.partial(
        pl.kernel, _sc_scatter_body,
        mesh=plsc.VectorSubcoreMesh(core_axis_name="c", subcore_axis_name="s"),
        out_type=[
            jax.ShapeDtypeStruct((_ROWS, 128), _f32),
            jax.ShapeDtypeStruct((_ROWS, 128), _f32),
        ],
        scratch_types=[
            pltpu.VMEM((_WPER,), _i32),
            pltpu.VMEM((_WPER, 128), _f32),
            pltpu.VMEM((_WPER, 128), _f32),
            pltpu.SemaphoreType.DMA,
            pltpu.SemaphoreType.DMA,
        ],
    )()
    xgp, vegp = sc_scatter(xp, vep, posg.reshape(_ROWS))
    xg = xgp[:, :T].reshape(B, N, T)
    veg = vegp[:, :D].reshape(B, N, D)

    x_pad = jnp.pad(xg, ((0, 0), (0, 0), (0, 2)))
    widx = jnp.arange(LP)[:, None] * STRIDE + jnp.arange(PATCH)[None, :]
    patches_tm = x_pad[:, :, widx].transpose(0, 2, 1, 3)

    w2 = lambda b, n: (0, 0)
    nb3 = lambda b, n: (b, 0, 0)
    tok, y_mix = pl.pallas_call(
        _mixer_body,
        grid=(B, NB),
        in_specs=[
            pl.BlockSpec((1, LP, RBLK, PATCH), lambda b, n: (b, 0, n, 0)),
            pl.BlockSpec((1, RBLK, D), lambda b, n: (b, n, 0)),
            pl.BlockSpec((1, 1, 1), nb3),
            pl.BlockSpec((PATCH, D), w2),
            pl.BlockSpec((1, D), w2),
            pl.BlockSpec((D, 2 * D), w2),
            pl.BlockSpec((PATCH, 2 * D), w2),
            pl.BlockSpec((2 * D, D), w2),
            pl.BlockSpec((L * D, PRED), w2),
        ],
        out_specs=[
            pl.BlockSpec((1, RBLK, D), lambda b, n: (b, n, 0)),
            pl.BlockSpec((1, RBLK, PRED), lambda b, n: (b, n, 0)),
        ],
        out_shape=[
            jax.ShapeDtypeStruct((B, N, D), _f32),
            jax.ShapeDtypeStruct((B, N, PRED), _f32),
        ],
    )(patches_tm, veg, nobs, W_patch, bp2, Wt1, ckT, Wt2, W_head)

    for lyr in range(NLAYERS):
        tok = pl.pallas_call(
            _enc_body,
            grid=(B, NB),
            in_specs=[
                pl.BlockSpec((1, N, D), nb3),
                pl.BlockSpec((1, 1, 1), nb3),
                pl.BlockSpec((D, D), w2),
                pl.BlockSpec((D, D), w2),
                pl.BlockSpec((D, D), w2),
                pl.BlockSpec((D, D), w2),
            ],
            out_specs=pl.BlockSpec((1, RBLK, D), lambda b, n: (b, n, 0)),
            out_shape=jax.ShapeDtypeStruct((B, N, D), _f32),
        )(tok, nobs, Wq[lyr], Wk[lyr], Wv[lyr], Wo[lyr])

    y_dec, q_sub, w_sub = pl.pallas_call(
        _dec_body,
        grid=(B, NB),
        in_specs=[
            pl.BlockSpec((1, N, D), nb3),
            pl.BlockSpec((1, RBLK, D), lambda b, n: (b, n, 0)),
            pl.BlockSpec((1, 1, 1), nb3),
            pl.BlockSpec((D, D), w2),
            pl.BlockSpec((D, D), w2),
            pl.BlockSpec((D, D), w2),
            pl.BlockSpec((D, D), w2),
            pl.BlockSpec((K, D), w2),
            pl.BlockSpec((D, K), w2),
            pl.BlockSpec((D, PRED), w2),
        ],
        out_specs=[
            pl.BlockSpec((1, RBLK, PRED), lambda b, n: (b, n, 0)),
            pl.BlockSpec((1, 1, D), nb3),
            pl.BlockSpec((1, 1, K), nb3),
        ],
        out_shape=[
            jax.ShapeDtypeStruct((B, N, PRED), _f32),
            jax.ShapeDtypeStruct((B, 1, D), _f32),
            jax.ShapeDtypeStruct((B, 1, K), _f32),
        ],
    )(tok, veg, nobs, Wq2, Wk2, Wv2, Wo2, C, CT, Whs)

    y_hat = pl.pallas_call(
        _fin_body,
        grid=(B,),
        in_specs=[
            pl.BlockSpec((1, N, 1), g3),
            pl.BlockSpec((1, N, PRED), g3),
            pl.BlockSpec((1, N, PRED), g3),
            pl.BlockSpec((1, PRED), g2),
        ],
        out_specs=pl.BlockSpec((1, N, PRED), g3),
        out_shape=jax.ShapeDtypeStruct((B, N, PRED), _f32),
    )(obs_col, y_mix, y_dec, bh2)

    return (y_hat, q_sub.reshape(B, D), w_sub.reshape(B, K))
